# baseline (speedup 1.0000x reference)
import functools

import jax
import jax.numpy as jnp
from jax import lax
from jax.experimental import pallas as pl
from jax.experimental.pallas import tpu as pltpu
from jax.experimental.pallas import tpu_sc as plsc

L = 16
PBLK = 128
NPB = 16
BH = 8


def _emb_kernel(idx4, tok_hbm, pos4, out5, idx_v, pos_v, rows2, out_v,
                sem_g, sem_o):
    wid = lax.axis_index("s") * 2 + lax.axis_index("c")
    pb = wid % NPB
    h = wid // NPB

    pltpu.sync_copy(idx4.at[h, pb], idx_v)
    pltpu.sync_copy(pos4.at[:, pb], pos_v)

    iota = lax.iota(jnp.int32, L)
    row_sel = [iota + (l0 * L) for l0 in range(PBLK // L)]

    def start_gather(k, buf):
        return pltpu.async_copy(tok_hbm.at[idx_v.at[k]], rows2.at[buf],
                                sem_g, priority=1)

    start_gather(0, 0)

    @pl.loop(0, BH)
    def _chunk(k):
        par = lax.rem(k, 2)
        pltpu.make_async_copy(tok_hbm.at[idx_v.at[k]], rows2.at[par],
                              sem_g).wait()

        @pl.when(k < BH - 1)
        def _():
            start_gather(k + 1, 1 - par)

        par_v = jnp.full((L,), 0, jnp.int32) + par
        for r in range(8):
            for s in range(8):
                d_v = jnp.full((L,), r * 8 + s, jnp.int32)
                for l0 in range(PBLK // L):
                    sl = pl.ds(l0 * L, L)
                    v = plsc.load_gather(rows2, [par_v, row_sel[l0], d_v])
                    out_v[r, s, sl] = v + pos_v[r, s, sl]

        pltpu.sync_copy(out_v, out5.at[h * BH + k, :, pb])


def kernel(idx, tok_table, pos_table):
    B, T = idx.shape
    V, D = tok_table.shape
    idx = idx.astype(jnp.int32)

    idx4 = idx.reshape(B // 8, 8, T // 128, 128).transpose(0, 2, 1, 3)
    pos4 = pos_table.T.reshape(D // 8, 8, T // 128, 128).transpose(0, 2, 1, 3)

    mesh = plsc.VectorSubcoreMesh(core_axis_name="c", subcore_axis_name="s")

    k = pl.kernel(
        _emb_kernel,
        out_type=jax.ShapeDtypeStruct((B, D // 8, T // 128, 8, 128),
                                      jnp.float32),
        mesh=mesh,
        scratch_types=[
            pltpu.VMEM((BH, PBLK), jnp.int32),
            pltpu.VMEM((D // 8, 8, PBLK), jnp.float32),
            pltpu.VMEM((2, PBLK, D), jnp.float32),
            pltpu.VMEM((D // 8, 8, PBLK), jnp.float32),
            pltpu.SemaphoreType.DMA,
            pltpu.SemaphoreType.DMA,
        ],
        compiler_params=pltpu.CompilerParams(use_tc_tiling_on_sc=False,
                                             needs_layout_passes=False),
    )
    out5 = k(idx4, tok_table, pos4)
    return out5.transpose(0, 2, 4, 1, 3).reshape(B, T, D)


# speedup vs baseline: 1.2990x; 1.2990x over previous
import jax
import jax.numpy as jnp
from jax import lax
from jax.experimental import pallas as pl
from jax.experimental.pallas import tpu as pltpu
from jax.experimental.pallas import tpu_sc as plsc

L = 16
PBLK = 128
NPB = 16
BH = 8
DP = 128
T_SEQ = 2048


def _emb_kernel(idx4, tokp, posp, out2, idx_v, rows2, sem_p, sem_g, sem_o):
    wid = lax.axis_index("s") * 2 + lax.axis_index("c")
    pb = wid % NPB
    h = wid // NPB
    p0 = pb * PBLK

    pltpu.sync_copy(idx4.at[h, pb], idx_v)

    def prefill(buf):
        return pltpu.async_copy(posp.at[pl.ds(p0, PBLK)], rows2.at[buf],
                                sem_p)

    def gather_add(k, buf):
        return pltpu.async_copy(tokp.at[idx_v.at[k]], rows2.at[buf], sem_g,
                                add=True)

    prefill(0)

    @pl.loop(0, BH)
    def _chunk(k):
        par = lax.rem(k, 2)
        pltpu.make_async_copy(posp.at[pl.ds(p0, PBLK)], rows2.at[par],
                              sem_p).wait()
        gather_add(k, par)
        pltpu.make_async_copy(tokp.at[idx_v.at[k]], rows2.at[par],
                              sem_g).wait()

        pltpu.async_copy(rows2.at[par],
                         out2.at[pl.ds((h * BH + k) * T_SEQ + p0, PBLK)],
                         sem_o)

        @pl.when(k >= 1)
        def _():
            pltpu.make_async_copy(rows2.at[0], out2.at[pl.ds(0, PBLK)],
                                  sem_o).wait()

        @pl.when(k < BH - 1)
        def _():
            prefill(1 - par)

    pltpu.make_async_copy(rows2.at[0], out2.at[pl.ds(0, PBLK)], sem_o).wait()


def kernel(idx, tok_table, pos_table):
    B, T = idx.shape
    V, D = tok_table.shape
    idx = idx.astype(jnp.int32)

    idx4 = idx.reshape(B // 8, 8, T // 128, 128).transpose(0, 2, 1, 3)
    tokp = jnp.pad(tok_table, ((0, 0), (0, DP - D)))
    posp = jnp.pad(pos_table, ((0, 0), (0, DP - D)))

    mesh = plsc.VectorSubcoreMesh(core_axis_name="c", subcore_axis_name="s")

    k = pl.kernel(
        _emb_kernel,
        out_type=jax.ShapeDtypeStruct((B * T, DP), jnp.float32),
        mesh=mesh,
        scratch_types=[
            pltpu.VMEM((BH, PBLK), jnp.int32),
            pltpu.VMEM((2, PBLK, DP), jnp.float32),
            pltpu.SemaphoreType.DMA,
            pltpu.SemaphoreType.DMA,
            pltpu.SemaphoreType.DMA,
        ],
        compiler_params=pltpu.CompilerParams(use_tc_tiling_on_sc=True),
    )
    out2 = k(idx4, tokp, posp)
    return out2.reshape(B, T, DP)[:, :, :D]


# speedup vs baseline: 1.4264x; 1.0981x over previous
import jax
import jax.numpy as jnp
from jax import lax
from jax.experimental import pallas as pl
from jax.experimental.pallas import tpu as pltpu
from jax.experimental.pallas import tpu_sc as plsc

L = 16
PBLK = 128
NPB = 16
BH = 8
CB = 2176
VHALF = 23 * CB


def _pairs_tc_kernel(a_ref, b_ref, out_ref):
    out_ref[...] = jnp.concatenate([a_ref[...].T, b_ref[...].T], axis=1)


def _emb_kernel(idx4, tok2, pos2, out3, idx_v, idx_g, pos_v, rows2,
                out_v, sem_g, sem_o):
    wid = lax.axis_index("s") * 2 + lax.axis_index("c")
    pb = wid % NPB
    h = wid // NPB

    pltpu.sync_copy(idx4.at[h, pb], idx_v)
    pltpu.sync_copy(pos2.at[pl.ds(pb * (PBLK // 2), PBLK // 2)], pos_v)

    for k in range(BH):
        for l0 in range(PBLK // L):
            sl = pl.ds(l0 * L, L)
            v = idx_v[k, sl]
            idx_g[k, sl] = jnp.where(v >= VHALF, v - VHALF, v)

    def start_gather(k, buf):
        return pltpu.async_copy(tok2.at[idx_g.at[k]], rows2.at[buf], sem_g)

    start_gather(0, 0)

    @pl.loop(0, BH)
    def _chunk(k):
        par = lax.rem(k, 2)
        pltpu.make_async_copy(tok2.at[idx_g.at[k]], rows2.at[par],
                              sem_g).wait()

        @pl.when(k < BH - 1)
        def _():
            start_gather(k + 1, 1 - par)

        @pl.loop(0, PBLK // L)
        def _grp(g):
            vg = idx_v[k, pl.ds(g * L, L)]
            pv = jnp.where(vg >= VHALF, 64, 0)
            for i in range(L):
                t = g * L + i
                p_row = g * (L // 2) + i // 2
                half = pv[i]
                for c in range(4):
                    sl_o = pl.ds((i % 2) * 64 + c * L, L)
                    out_v[par, p_row, sl_o] = (
                        rows2[par, t, pl.ds(half + c * L, L)]
                        + pos_v[p_row, sl_o])

        pltpu.async_copy(out_v.at[par],
                         out3.at[h * BH + k, pl.ds(pb * (PBLK // 2),
                                                   PBLK // 2)],
                         sem_o)

        @pl.when(k >= 1)
        def _():
            pltpu.make_async_copy(out_v.at[0],
                                  out3.at[0, pl.ds(0, PBLK // 2)],
                                  sem_o).wait()

    pltpu.make_async_copy(out_v.at[0], out3.at[0, pl.ds(0, PBLK // 2)],
                          sem_o).wait()


def kernel(idx, tok_table, pos_table):
    B, T = idx.shape
    V, D = tok_table.shape
    idx = idx.astype(jnp.int32)

    idx4 = idx.reshape(B // 8, 8, T // 128, 128).transpose(0, 2, 1, 3)
    tokT = tok_table.T
    pos2 = pos_table.reshape(T // 2, 2 * D)

    nblk = VHALF // CB
    tok2 = pl.pallas_call(
        _pairs_tc_kernel,
        grid=(nblk,),
        in_specs=[pl.BlockSpec((D, CB), lambda i: (0, i)),
                  pl.BlockSpec((D, CB), lambda i: (0, i + VHALF // CB))],
        out_specs=pl.BlockSpec((CB, 2 * D), lambda i: (i, 0)),
        out_shape=jax.ShapeDtypeStruct((VHALF, 2 * D), jnp.float32),
    )(tokT, tokT)

    mesh = plsc.VectorSubcoreMesh(core_axis_name="c", subcore_axis_name="s")

    k = pl.kernel(
        _emb_kernel,
        out_type=jax.ShapeDtypeStruct((B, T // 2, 2 * D), jnp.float32),
        mesh=mesh,
        scratch_types=[
            pltpu.VMEM((BH, PBLK), jnp.int32),
            pltpu.VMEM((BH, PBLK), jnp.int32),
            pltpu.VMEM((PBLK // 2, 2 * D), jnp.float32),
            pltpu.VMEM((2, PBLK, 2 * D), jnp.float32),
            pltpu.VMEM((2, PBLK // 2, 2 * D), jnp.float32),
            pltpu.SemaphoreType.DMA,
            pltpu.SemaphoreType.DMA,
        ],
        compiler_params=pltpu.CompilerParams(use_tc_tiling_on_sc=False),
    )
    out3 = k(idx4, tok2, pos2)
    return out3.reshape(B, T, D)


# speedup vs baseline: 1.5193x; 1.0651x over previous
import jax
import jax.numpy as jnp
from jax import lax
from jax.experimental import pallas as pl
from jax.experimental.pallas import tpu as pltpu
from jax.experimental.pallas import tpu_sc as plsc

L = 16
PBLK = 128
NPB = 16
BH = 8
CB = 2176
VHALF = 23 * CB


def _pairs_tc_kernel(a_ref, b_ref, out_ref):
    out_ref[...] = jnp.concatenate([a_ref[...].T, b_ref[...].T], axis=1)


def _emb_kernel(idx4, tokd, posf, out, idx_v, idx_g, rows2,
                sem_p, sem_g, sem_o):
    wid = lax.axis_index("s") * 2 + lax.axis_index("c")
    pb = wid % NPB
    h = wid // NPB
    p0 = pb * PBLK

    pltpu.sync_copy(idx4.at[h, pb], idx_v)

    for k in range(BH):
        for l0 in range(PBLK // L):
            sl = pl.ds(l0 * L, L)
            v = idx_v[k, sl]
            v2 = v + v
            idx_g[k, sl] = jnp.where(v < VHALF, v2, v2 - (2 * VHALF - 1))

    def prefill(buf):
        return pltpu.async_copy(posf.at[pl.ds(p0, PBLK)], rows2.at[buf],
                                sem_p)

    prefill(0)

    @pl.loop(0, BH)
    def _chunk(k):
        par = lax.rem(k, 2)
        pltpu.make_async_copy(posf.at[pl.ds(p0, PBLK)], rows2.at[par],
                              sem_p).wait()
        pltpu.async_copy(tokd.at[idx_g.at[k]], rows2.at[par], sem_g,
                         add=True)
        pltpu.make_async_copy(tokd.at[idx_g.at[k]], rows2.at[par],
                              sem_g).wait()

        pltpu.async_copy(rows2.at[par], out.at[h * BH + k, pl.ds(p0, PBLK)],
                         sem_o)

        @pl.when(k >= 1)
        def _():
            pltpu.make_async_copy(rows2.at[0], out.at[0, pl.ds(0, PBLK)],
                                  sem_o).wait()

        @pl.when(k < BH - 1)
        def _():
            prefill(1 - par)

    pltpu.make_async_copy(rows2.at[0], out.at[0, pl.ds(0, PBLK)],
                          sem_o).wait()


def kernel(idx, tok_table, pos_table):
    B, T = idx.shape
    V, D = tok_table.shape
    idx = idx.astype(jnp.int32)

    idx4 = idx.reshape(B // 8, 8, T // 128, 128).transpose(0, 2, 1, 3)
    tokT = tok_table.T

    nblk = VHALF // CB
    tok2 = pl.pallas_call(
        _pairs_tc_kernel,
        grid=(nblk,),
        in_specs=[pl.BlockSpec((D, CB), lambda i: (0, i)),
                  pl.BlockSpec((D, CB), lambda i: (0, i + VHALF // CB))],
        out_specs=pl.BlockSpec((CB, 2 * D), lambda i: (i, 0)),
        out_shape=jax.ShapeDtypeStruct((VHALF, 2 * D), jnp.float32),
    )(tokT, tokT)
    tokd = tok2.reshape(2 * VHALF, D)

    mesh = plsc.VectorSubcoreMesh(core_axis_name="c", subcore_axis_name="s")

    k = pl.kernel(
        _emb_kernel,
        out_type=jax.ShapeDtypeStruct((B, T, D), jnp.float32),
        mesh=mesh,
        scratch_types=[
            pltpu.VMEM((BH, PBLK), jnp.int32),
            pltpu.VMEM((BH, PBLK), jnp.int32),
            pltpu.VMEM((2, PBLK, D), jnp.float32),
            pltpu.SemaphoreType.DMA,
            pltpu.SemaphoreType.DMA,
            pltpu.SemaphoreType.DMA,
        ],
        compiler_params=pltpu.CompilerParams(use_tc_tiling_on_sc=False),
    )
    return k(idx4, tokd, pos_table)


# speedup vs baseline: 1.5675x; 1.0317x over previous
import jax
import jax.numpy as jnp
from jax import lax
from jax.experimental import pallas as pl
from jax.experimental.pallas import tpu as pltpu
from jax.experimental.pallas import tpu_sc as plsc

L = 16
PBLK = 128
NPB = 16
BH = 8
CB = 2944
VHALF = 17 * CB


def _pairs_tc_kernel(a_ref, b_ref, out_ref):
    out_ref[...] = jnp.concatenate([a_ref[...].T, b_ref[...].T], axis=1)


def _emb_kernel(idx4, tokd, posf, out, idx_v, idx_g, rows2,
                sem_p, sem_g, sem_o):
    wid = lax.axis_index("s") * 2 + lax.axis_index("c")
    pb = wid % NPB
    h = wid // NPB
    p0 = pb * PBLK

    pltpu.sync_copy(idx4.at[h, pb], idx_v)

    for k in range(BH):
        for l0 in range(PBLK // L):
            sl = pl.ds(l0 * L, L)
            v = idx_v[k, sl]
            v2 = v + v
            idx_g[k, sl] = jnp.where(v < VHALF, v2, v2 - (2 * VHALF - 1))

    def prefill(buf):
        return pltpu.async_copy(posf.at[pl.ds(p0, PBLK)], rows2.at[buf],
                                sem_p)

    prefill(0)

    @pl.loop(0, BH)
    def _chunk(k):
        par = lax.rem(k, 2)
        pltpu.make_async_copy(posf.at[pl.ds(p0, PBLK)], rows2.at[par],
                              sem_p).wait()
        pltpu.async_copy(tokd.at[idx_g.at[k]], rows2.at[par], sem_g,
                         add=True)
        pltpu.make_async_copy(tokd.at[idx_g.at[k]], rows2.at[par],
                              sem_g).wait()

        pltpu.async_copy(rows2.at[par], out.at[h * BH + k, pl.ds(p0, PBLK)],
                         sem_o)

        @pl.when(k >= 1)
        def _():
            pltpu.make_async_copy(rows2.at[0], out.at[0, pl.ds(0, PBLK)],
                                  sem_o).wait()

        @pl.when(k < BH - 1)
        def _():
            prefill(1 - par)

    pltpu.make_async_copy(rows2.at[0], out.at[0, pl.ds(0, PBLK)],
                          sem_o).wait()


def kernel(idx, tok_table, pos_table):
    B, T = idx.shape
    V, D = tok_table.shape
    idx = idx.astype(jnp.int32)

    idx4 = idx.reshape(B // 8, 8, T // 128, 128).transpose(0, 2, 1, 3)
    tokT = tok_table.T

    nblk = VHALF // CB
    tok2 = pl.pallas_call(
        _pairs_tc_kernel,
        grid=(nblk,),
        in_specs=[pl.BlockSpec((D, CB), lambda i: (0, i)),
                  pl.BlockSpec((D, CB), lambda i: (0, i + VHALF // CB))],
        out_specs=pl.BlockSpec((CB, 2 * D), lambda i: (i, 0)),
        out_shape=jax.ShapeDtypeStruct((VHALF, 2 * D), jnp.float32),
    )(tokT, tokT)
    tokd = tok2.reshape(2 * VHALF, D)

    mesh = plsc.VectorSubcoreMesh(core_axis_name="c", subcore_axis_name="s")

    k = pl.kernel(
        _emb_kernel,
        out_type=jax.ShapeDtypeStruct((B, T, D), jnp.float32),
        mesh=mesh,
        scratch_types=[
            pltpu.VMEM((BH, PBLK), jnp.int32),
            pltpu.VMEM((BH, PBLK), jnp.int32),
            pltpu.VMEM((2, PBLK, D), jnp.float32),
            pltpu.SemaphoreType.DMA,
            pltpu.SemaphoreType.DMA,
            pltpu.SemaphoreType.DMA,
        ],
        compiler_params=pltpu.CompilerParams(use_tc_tiling_on_sc=False),
    )
    return k(idx4, tokd, pos_table)


# speedup vs baseline: 1.7086x; 1.0900x over previous
import jax
import jax.numpy as jnp
from jax import lax
from jax.experimental import pallas as pl
from jax.experimental.pallas import tpu as pltpu
from jax.experimental.pallas import tpu_sc as plsc

L = 16
PBLK = 128
NPB = 16
BH = 8
CB = 2944
VHALF = 17 * CB
NBUF = 4


def _pairs_tc_kernel(a_ref, b_ref, out_ref):
    out_ref[...] = jnp.concatenate([a_ref[...].T, b_ref[...].T], axis=1)


def _emb_kernel(idx4, tokd, posf, out, idx_v, idx_g, rows2,
                sem_p, sem_g, sem_o):
    wid = lax.axis_index("s") * 2 + lax.axis_index("c")
    pb = wid % NPB
    h = wid // NPB
    p0 = pb * PBLK

    pltpu.sync_copy(idx4.at[h, pb], idx_v)

    for k in range(BH):
        for l0 in range(PBLK // L):
            sl = pl.ds(l0 * L, L)
            v = idx_v[k, sl]
            v2 = v + v
            idx_g[k, sl] = jnp.where(v < VHALF, v2, v2 - (2 * VHALF - 1))

    def prefill(buf):
        return pltpu.async_copy(posf.at[pl.ds(p0, PBLK)], rows2.at[buf],
                                sem_p)

    def gather(k, buf):
        return pltpu.async_copy(tokd.at[idx_g.at[k]], rows2.at[buf], sem_g,
                                add=True)

    prefill(0)
    prefill(1)
    prefill(2)
    pltpu.make_async_copy(posf.at[pl.ds(p0, PBLK)], rows2.at[0],
                          sem_p).wait()
    gather(0, 0)

    @pl.loop(0, BH)
    def _chunk(k):
        par = lax.rem(k, NBUF)

        @pl.when(k < BH - 1)
        def _():
            pltpu.make_async_copy(posf.at[pl.ds(p0, PBLK)], rows2.at[0],
                                  sem_p).wait()
            gather(k + 1, lax.rem(k + 1, NBUF))

        pltpu.make_async_copy(tokd.at[idx_g.at[k]], rows2.at[par],
                              sem_g).wait()
        pltpu.async_copy(rows2.at[par], out.at[h * BH + k, pl.ds(p0, PBLK)],
                         sem_o)

        @pl.when(k >= 1)
        def _():
            pltpu.make_async_copy(rows2.at[0], out.at[0, pl.ds(0, PBLK)],
                                  sem_o).wait()

        @pl.when(k < BH - 3)
        def _():
            prefill(lax.rem(k + 3, NBUF))

    pltpu.make_async_copy(rows2.at[0], out.at[0, pl.ds(0, PBLK)],
                          sem_o).wait()


def kernel(idx, tok_table, pos_table):
    B, T = idx.shape
    V, D = tok_table.shape
    idx = idx.astype(jnp.int32)

    idx4 = idx.reshape(B // 8, 8, T // 128, 128).transpose(0, 2, 1, 3)
    tokT = tok_table.T

    nblk = VHALF // CB
    tok2 = pl.pallas_call(
        _pairs_tc_kernel,
        grid=(nblk,),
        in_specs=[pl.BlockSpec((D, CB), lambda i: (0, i)),
                  pl.BlockSpec((D, CB), lambda i: (0, i + VHALF // CB))],
        out_specs=pl.BlockSpec((CB, 2 * D), lambda i: (i, 0)),
        out_shape=jax.ShapeDtypeStruct((VHALF, 2 * D), jnp.float32),
        compiler_params=pltpu.CompilerParams(
            dimension_semantics=("arbitrary",)),
    )(tokT, tokT)
    tokd = tok2.reshape(2 * VHALF, D)

    mesh = plsc.VectorSubcoreMesh(core_axis_name="c", subcore_axis_name="s")

    k = pl.kernel(
        _emb_kernel,
        out_type=jax.ShapeDtypeStruct((B, T, D), jnp.float32),
        mesh=mesh,
        scratch_types=[
            pltpu.VMEM((BH, PBLK), jnp.int32),
            pltpu.VMEM((BH, PBLK), jnp.int32),
            pltpu.VMEM((NBUF, PBLK, D), jnp.float32),
            pltpu.SemaphoreType.DMA,
            pltpu.SemaphoreType.DMA,
            pltpu.SemaphoreType.DMA,
        ],
        compiler_params=pltpu.CompilerParams(use_tc_tiling_on_sc=False),
    )
    return k(idx4, tokd, pos_table)


# speedup vs baseline: 2.0266x; 1.1861x over previous
import jax
import jax.numpy as jnp
from jax import lax
from jax.experimental import pallas as pl
from jax.experimental.pallas import tpu as pltpu
from jax.experimental.pallas import tpu_sc as plsc

L = 16
PBLK = 128
NPB = 16
BH = 8
CB = 2944
VHALF = 17 * CB
NBUF = 4
DP = 128


def _pairs_tc_kernel(a_ref, b_ref, out_ref):
    out_ref[...] = jnp.concatenate([a_ref[...].T, b_ref[...].T], axis=1)


def _finish_tc_kernel(x_ref, posT_ref, o_ref):
    o_ref[0] = x_ref[...][:, :posT_ref.shape[0]].T + posT_ref[...]


def _emb_kernel(idx4, tokd, out, idx_v, idx_g, rows, sem_g, sem_o):
    wid = lax.axis_index("s") * 2 + lax.axis_index("c")
    pb = wid % NPB
    h = wid // NPB
    p0 = pb * PBLK

    pltpu.sync_copy(idx4.at[h, pb], idx_v)

    for k in range(BH):
        for l0 in range(PBLK // L):
            sl = pl.ds(l0 * L, L)
            v = idx_v[k, sl]
            v2 = v + v
            idx_g[k, sl] = jnp.where(v < VHALF, v2, v2 - (2 * VHALF - 1))

    def gather(k, buf):
        return pltpu.async_copy(tokd.at[idx_g.at[k]], rows.at[buf], sem_g)

    def out_slot(k):
        return out.at[pl.ds((h * BH + k) * 2048 + p0, PBLK), pl.ds(0, 64)]

    gather(0, 0)

    @pl.loop(0, BH)
    def _chunk(k):
        par = lax.rem(k, NBUF)

        @pl.when(k < BH - 1)
        def _():
            gather(k + 1, lax.rem(k + 1, NBUF))

        pltpu.make_async_copy(tokd.at[idx_g.at[k]], rows.at[par],
                              sem_g).wait()
        pltpu.async_copy(rows.at[par], out_slot(k), sem_o)

        @pl.when(k >= NBUF - 2)
        def _():
            pltpu.make_async_copy(rows.at[0], out_slot(0), sem_o).wait()

    for _ in range(NBUF - 2):
        pltpu.make_async_copy(rows.at[0], out_slot(0), sem_o).wait()


def kernel(idx, tok_table, pos_table):
    B, T = idx.shape
    V, D = tok_table.shape
    idx = idx.astype(jnp.int32)

    idx4 = idx.reshape(B // 8, 8, T // 128, 128).transpose(0, 2, 1, 3)
    tokT = tok_table.T
    posT = pos_table.T

    tok2 = pl.pallas_call(
        _pairs_tc_kernel,
        grid=(VHALF // CB,),
        in_specs=[pl.BlockSpec((D, CB), lambda i: (0, i)),
                  pl.BlockSpec((D, CB), lambda i: (0, i + VHALF // CB))],
        out_specs=pl.BlockSpec((CB, 2 * D), lambda i: (i, 0)),
        out_shape=jax.ShapeDtypeStruct((VHALF, 2 * D), jnp.float32),
        compiler_params=pltpu.CompilerParams(
            dimension_semantics=("arbitrary",)),
    )(tokT, tokT)
    tokd = tok2.reshape(2 * VHALF, D)

    mesh = plsc.VectorSubcoreMesh(core_axis_name="c", subcore_axis_name="s")

    gathered = pl.kernel(
        _emb_kernel,
        out_type=jax.ShapeDtypeStruct((B * T, DP), jnp.float32),
        mesh=mesh,
        scratch_types=[
            pltpu.VMEM((BH, PBLK), jnp.int32),
            pltpu.VMEM((BH, PBLK), jnp.int32),
            pltpu.VMEM((NBUF, PBLK, D), jnp.float32),
            pltpu.SemaphoreType.DMA,
            pltpu.SemaphoreType.DMA,
        ],
        compiler_params=pltpu.CompilerParams(use_tc_tiling_on_sc=False),
    )(idx4, tokd)

    outT = pl.pallas_call(
        _finish_tc_kernel,
        grid=(B,),
        in_specs=[pl.BlockSpec((T, DP), lambda b: (b, 0)),
                  pl.BlockSpec((D, T), lambda b: (0, 0))],
        out_specs=pl.BlockSpec((1, D, T), lambda b: (b, 0, 0)),
        out_shape=jax.ShapeDtypeStruct((B, D, T), jnp.float32),
        compiler_params=pltpu.CompilerParams(
            dimension_semantics=("arbitrary",)),
    )(gathered, posT)
    return jnp.swapaxes(outT, 1, 2)
